# baseline (speedup 1.0000x reference)
import functools

import jax
import jax.numpy as jnp
from jax import lax
from jax.experimental import pallas as pl
from jax.experimental.pallas import tpu as pltpu
from jax.experimental.pallas import tpu_sc as plsc

N_NODES = 10000
N_EDGES = 320000
D = 128
L = 16
C = 80


def _edge_partials(h, src, dst):
    info = plsc.get_sparse_core_info()
    nc, ns = info.num_cores, info.num_subcores
    nw = nc * ns
    ew = N_EDGES // nw
    n_chunks = ew // C

    @functools.partial(
        pl.kernel,
        out_type=jax.ShapeDtypeStruct((N_EDGES, L), jnp.float32),
        mesh=plsc.VectorSubcoreMesh(core_axis_name="c", subcore_axis_name="s"),
        scratch_types=[
            pltpu.VMEM((ew,), jnp.int32),
            pltpu.VMEM((ew,), jnp.int32),
            pltpu.VMEM((2 * C, D), jnp.float32),
            pltpu.VMEM((2 * C, D), jnp.float32),
            pltpu.VMEM((2 * C, L), jnp.float32),
            pltpu.SemaphoreType.DMA((2,)),
            pltpu.SemaphoreType.DMA((2,)),
        ],
    )
    def k(h_ref, src_ref, dst_ref, out_ref,
          idx_s, idx_d, rows_s, rows_d, parts, gsem, wsem):
        wid = lax.axis_index("s") * nc + lax.axis_index("c")
        ebase = pl.multiple_of(wid * ew, 8)
        pltpu.sync_copy(src_ref.at[pl.ds(ebase, ew)], idx_s)
        pltpu.sync_copy(dst_ref.at[pl.ds(ebase, ew)], idx_d)

        def fire(ch, slot):
            eoff = pl.multiple_of(ch * C, 8)
            poff = pl.multiple_of(slot * C, 8)
            s = gsem.at[slot]
            pltpu.async_copy(h_ref.at[idx_s.at[pl.ds(eoff, C)]],
                             rows_s.at[pl.ds(poff, C)], s)
            pltpu.async_copy(h_ref.at[idx_d.at[pl.ds(eoff, C)]],
                             rows_d.at[pl.ds(poff, C)], s)

        def drain(ch):
            poff = pl.multiple_of((ch & 1) * C, 8)
            s = gsem.at[ch & 1]
            pltpu.make_async_copy(h_ref.at[idx_s.at[pl.ds(0, C)]],
                                  rows_s.at[pl.ds(poff, C)], s).wait()
            pltpu.make_async_copy(h_ref.at[idx_d.at[pl.ds(0, C)]],
                                  rows_d.at[pl.ds(poff, C)], s).wait()

        def drain_write(ch):
            poff = pl.multiple_of((ch & 1) * C, 8)
            pltpu.make_async_copy(
                parts.at[pl.ds(poff, C)],
                out_ref.at[pl.ds(ebase, C)],
                wsem.at[ch & 1]).wait()

        fire(0, 0)

        def body(ch, carry):
            fire(jnp.minimum(ch + 1, n_chunks - 1), (ch + 1) & 1)
            drain(ch)

            @pl.when(ch >= 2)
            def _():
                drain_write(ch)

            poff = (ch & 1) * C

            def group(eg, carry2):
                base_e = poff + eg * L
                for s in range(L):
                    ei = base_e + s
                    prods = [rows_s[ei, pl.ds(j * L, L)]
                             * rows_d[ei, pl.ds(j * L, L)]
                             for j in range(D // L)]
                    while len(prods) > 1:
                        prods = [prods[2 * i] + prods[2 * i + 1]
                                 for i in range(len(prods) // 2)]
                    parts[ei, pl.ds(0, L)] = prods[0]
                return carry2

            lax.fori_loop(0, C // L, group, 0)
            pltpu.async_copy(parts.at[pl.ds(poff, C)],
                             out_ref.at[pl.ds(ebase + ch * C, C)],
                             wsem.at[ch & 1])
            return carry

        lax.fori_loop(0, n_chunks, body, 0)
        drain(n_chunks)
        drain_write(n_chunks - 2)
        drain_write(n_chunks - 1)

    return k(h, src, dst)


_BLK = 4000


def _row_sum16(parts):
    def body(x_ref, o_ref):
        o_ref[...] = jnp.sum(x_ref[...], axis=1, keepdims=True)

    return pl.pallas_call(
        body,
        grid=(N_EDGES // _BLK,),
        in_specs=[pl.BlockSpec((_BLK, L), lambda i: (i, 0))],
        out_specs=pl.BlockSpec((_BLK, 1), lambda i: (i, 0)),
        out_shape=jax.ShapeDtypeStruct((N_EDGES, 1), jnp.float32),
    )(parts)


def kernel(h, edge_index):
    ei = edge_index.astype(jnp.int32)
    parts = _edge_partials(h, ei[0], ei[1])
    return _row_sum16(parts)
